# baseline (speedup 1.0000x reference)
import functools

import jax
import jax.numpy as jnp
from jax import lax
from jax.experimental import pallas as pl
from jax.experimental.pallas import tpu as pltpu
from jax.experimental.pallas import tpu_sc as plsc

_CACHE_FRAMES = 256
_NEW_FRAMES = 64


def _sc_store(idx32, kf, vf, k_cache, v_cache):
    mesh = plsc.ScalarSubcoreMesh(axis_name="core", num_cores=2)

    @functools.partial(
        pl.kernel,
        out_type=(
            jax.ShapeDtypeStruct(k_cache.shape, k_cache.dtype),
            jax.ShapeDtypeStruct(v_cache.shape, v_cache.dtype),
        ),
        mesh=mesh,
        scratch_types=[
            pltpu.SMEM((_NEW_FRAMES,), jnp.int32),
            pltpu.SMEM((_CACHE_FRAMES,), jnp.int32),
            pltpu.SemaphoreType.DMA,
            pltpu.SemaphoreType.DMA,
        ],
    )
    def store_kernel(idx_hbm, kf_hbm, vf_hbm, kc_hbm, vc_hbm,
                     ok_hbm, ov_hbm, idx_s, src_s, idx_sem, sem):
        core = lax.axis_index("core")
        pltpu.async_copy(idx_hbm, idx_s, idx_sem).wait()

        @pl.loop(0, _CACHE_FRAMES)
        def _(j):
            src_s[j] = -1

        @pl.loop(0, _NEW_FRAMES)
        def _(i):
            src_s[idx_s[i]] = i

        def route(new_hbm, cache_hbm, out_hbm):
            @pl.loop(0, _CACHE_FRAMES)
            def _(j):
                s = src_s[j]

                @pl.when(s >= 0)
                def _():
                    pltpu.async_copy(new_hbm.at[s], out_hbm.at[j], sem)

                @pl.when(s < 0)
                def _():
                    pltpu.async_copy(cache_hbm.at[j], out_hbm.at[j], sem)

            @pl.loop(0, _CACHE_FRAMES)
            def _(j):
                pltpu.make_async_copy(cache_hbm.at[0], out_hbm.at[0], sem).wait()

        @pl.when(core == 0)
        def _():
            route(kf_hbm, kc_hbm, ok_hbm)

        @pl.when(core == 1)
        def _():
            route(vf_hbm, vc_hbm, ov_hbm)

    return store_kernel(idx32, kf, vf, k_cache, v_cache)


def kernel(k, v, idx, k_cache, v_cache):
    idx32 = idx.astype(jnp.int32) % _CACHE_FRAMES
    return _sc_store(idx32, k[0], v[0], k_cache, v_cache)


# speedup vs baseline: 1.0009x; 1.0009x over previous
import jax
import jax.numpy as jnp
from jax import lax
from jax.experimental import pallas as pl
from jax.experimental.pallas import tpu as pltpu

_CACHE_FRAMES = 256
_NEW_FRAMES = 64

_ANY = pl.ANY
_SMEM = pltpu.MemorySpace.SMEM


def _store_body(idx_s, kf_h, vf_h, kc_h, vc_h, ok_h, ov_h, src_s, sem):
    def init(j, c):
        src_s[j] = -1
        return c
    lax.fori_loop(0, _CACHE_FRAMES, init, 0, unroll=8)

    def setmap(i, c):
        src_s[idx_s[i]] = i
        return c
    lax.fori_loop(0, _NEW_FRAMES, setmap, 0, unroll=8)

    def issue(j, c):
        s = src_s[j]

        @pl.when(s >= 0)
        def _():
            pltpu.make_async_copy(kf_h.at[s], ok_h.at[j], sem).start()
            pltpu.make_async_copy(vf_h.at[s], ov_h.at[j], sem).start()

        @pl.when(s < 0)
        def _():
            pltpu.make_async_copy(kc_h.at[j], ok_h.at[j], sem).start()
            pltpu.make_async_copy(vc_h.at[j], ov_h.at[j], sem).start()
        return c
    lax.fori_loop(0, _CACHE_FRAMES, issue, 0, unroll=4)

    def drain(j, c):
        pltpu.make_async_copy(kc_h.at[0], ok_h.at[0], sem).wait()
        pltpu.make_async_copy(vc_h.at[0], ov_h.at[0], sem).wait()
        return c
    lax.fori_loop(0, _CACHE_FRAMES, drain, 0, unroll=4)


def kernel(k, v, idx, k_cache, v_cache):
    idx32 = idx.astype(jnp.int32) % _CACHE_FRAMES
    out = pl.pallas_call(
        _store_body,
        out_shape=(
            jax.ShapeDtypeStruct(k_cache.shape, k_cache.dtype),
            jax.ShapeDtypeStruct(v_cache.shape, v_cache.dtype),
        ),
        in_specs=[
            pl.BlockSpec(memory_space=_SMEM),
            pl.BlockSpec(memory_space=_ANY),
            pl.BlockSpec(memory_space=_ANY),
            pl.BlockSpec(memory_space=_ANY),
            pl.BlockSpec(memory_space=_ANY),
        ],
        out_specs=(
            pl.BlockSpec(memory_space=_ANY),
            pl.BlockSpec(memory_space=_ANY),
        ),
        scratch_shapes=[
            pltpu.SMEM((_CACHE_FRAMES,), jnp.int32),
            pltpu.SemaphoreType.DMA,
        ],
    )(idx32, k[0], v[0], k_cache, v_cache)
    return out


# speedup vs baseline: 14.4716x; 14.4587x over previous
import jax
import jax.numpy as jnp
from jax import lax
from jax.experimental import pallas as pl
from jax.experimental.pallas import tpu as pltpu

_CACHE_FRAMES = 256
_NEW_FRAMES = 64
_FRAME = (1, 128, 16, 64)


def _body(src_r, ksel_r, csel_r, kc_r, vc_r, kf_r, vf_r, ok_r, ov_r):
    s = src_r[pl.program_id(0)]

    @pl.when(s >= 0)
    def _():
        ok_r[...] = kf_r[...]
        ov_r[...] = vf_r[...]

    @pl.when(s < 0)
    def _():
        ok_r[...] = kc_r[...]
        ov_r[...] = vc_r[...]


def kernel(k, v, idx, k_cache, v_cache):
    idx32 = idx.astype(jnp.int32) % _CACHE_FRAMES
    src = jnp.full((_CACHE_FRAMES,), -1, jnp.int32).at[idx32].max(
        jnp.arange(_NEW_FRAMES, dtype=jnp.int32))
    has = src >= 0
    jpos = jnp.arange(_CACHE_FRAMES, dtype=jnp.int32)
    lastk = lax.cummax(jnp.where(has, jpos, -1))
    ksel = jnp.take(src, jnp.maximum(lastk, 0))
    ksel = jnp.maximum(ksel, 0)
    lastc = lax.cummax(jnp.where(has, -1, jpos))
    csel = jnp.maximum(lastc, 0)

    grid_spec = pltpu.PrefetchScalarGridSpec(
        num_scalar_prefetch=3,
        grid=(_CACHE_FRAMES,),
        in_specs=[
            pl.BlockSpec(_FRAME, lambda j, src, ksel, csel: (csel[j], 0, 0, 0)),
            pl.BlockSpec(_FRAME, lambda j, src, ksel, csel: (csel[j], 0, 0, 0)),
            pl.BlockSpec(_FRAME, lambda j, src, ksel, csel: (ksel[j], 0, 0, 0)),
            pl.BlockSpec(_FRAME, lambda j, src, ksel, csel: (ksel[j], 0, 0, 0)),
        ],
        out_specs=[
            pl.BlockSpec(_FRAME, lambda j, src, ksel, csel: (j, 0, 0, 0)),
            pl.BlockSpec(_FRAME, lambda j, src, ksel, csel: (j, 0, 0, 0)),
        ],
    )
    out = pl.pallas_call(
        _body,
        grid_spec=grid_spec,
        out_shape=(
            jax.ShapeDtypeStruct(k_cache.shape, k_cache.dtype),
            jax.ShapeDtypeStruct(v_cache.shape, v_cache.dtype),
        ),
        compiler_params=pltpu.CompilerParams(
            dimension_semantics=("arbitrary",),
        ),
    )(src, ksel, csel, k_cache, v_cache, k[0], v[0])
    return out


# speedup vs baseline: 21.1871x; 1.4640x over previous
import jax
import jax.numpy as jnp
from jax import lax
from jax.experimental import pallas as pl
from jax.experimental.pallas import tpu as pltpu

_CACHE_FRAMES = 256
_NEW_FRAMES = 64
_TOK = 128
_D = 16 * 64
_FRAME = (1, _TOK, _D)


def _body(src_r, ksel_r, csel_r, kc_r, vc_r, kf_r, vf_r, ok_r, ov_r):
    s = src_r[pl.program_id(0)]

    @pl.when(s >= 0)
    def _():
        ok_r[...] = kf_r[...]
        ov_r[...] = vf_r[...]

    @pl.when(s < 0)
    def _():
        ok_r[...] = kc_r[...]
        ov_r[...] = vc_r[...]


def kernel(k, v, idx, k_cache, v_cache):
    idx32 = idx.astype(jnp.int32) % _CACHE_FRAMES
    src = jnp.full((_CACHE_FRAMES,), -1, jnp.int32).at[idx32].max(
        jnp.arange(_NEW_FRAMES, dtype=jnp.int32))
    has = src >= 0
    jpos = jnp.arange(_CACHE_FRAMES, dtype=jnp.int32)
    lastk = lax.cummax(jnp.where(has, jpos, -1))
    ksel = jnp.take(src, jnp.maximum(lastk, 0))
    ksel = jnp.maximum(ksel, 0)
    lastc = lax.cummax(jnp.where(has, -1, jpos))
    csel = jnp.maximum(lastc, 0)

    grid_spec = pltpu.PrefetchScalarGridSpec(
        num_scalar_prefetch=3,
        grid=(_CACHE_FRAMES,),
        in_specs=[
            pl.BlockSpec(_FRAME, lambda j, src, ksel, csel: (csel[j], 0, 0)),
            pl.BlockSpec(_FRAME, lambda j, src, ksel, csel: (csel[j], 0, 0)),
            pl.BlockSpec(_FRAME, lambda j, src, ksel, csel: (ksel[j], 0, 0)),
            pl.BlockSpec(_FRAME, lambda j, src, ksel, csel: (ksel[j], 0, 0)),
        ],
        out_specs=[
            pl.BlockSpec(_FRAME, lambda j, src, ksel, csel: (j, 0, 0)),
            pl.BlockSpec(_FRAME, lambda j, src, ksel, csel: (j, 0, 0)),
        ],
    )
    out_k, out_v = pl.pallas_call(
        _body,
        grid_spec=grid_spec,
        out_shape=(
            jax.ShapeDtypeStruct((_CACHE_FRAMES, _TOK, _D), k_cache.dtype),
            jax.ShapeDtypeStruct((_CACHE_FRAMES, _TOK, _D), v_cache.dtype),
        ),
        compiler_params=pltpu.CompilerParams(
            dimension_semantics=("arbitrary",),
        ),
    )(src, ksel, csel,
      k_cache.reshape(_CACHE_FRAMES, _TOK, _D),
      v_cache.reshape(_CACHE_FRAMES, _TOK, _D),
      k.reshape(_NEW_FRAMES, _TOK, _D),
      v.reshape(_NEW_FRAMES, _TOK, _D))
    return out_k.reshape(k_cache.shape), out_v.reshape(v_cache.shape)


# speedup vs baseline: 24.5584x; 1.1591x over previous
import jax
import jax.numpy as jnp
from jax import lax
from jax.experimental import pallas as pl
from jax.experimental.pallas import tpu as pltpu

_CACHE_FRAMES = 256
_NEW_FRAMES = 64
_TOK = 128
_D = 16 * 64

_NBUF = 8
_LOOKAHEAD = 6

_ANY = pl.ANY
_SMEM = pltpu.MemorySpace.SMEM


def _store_body(idx_s, kf_h, vf_h, kc_h, vc_h, ok_h, ov_h,
                src_s, kbuf, vbuf, kin_sems, vin_sems, kout_sems, vout_sems):
    def init(j, c):
        src_s[j] = -1
        return c
    lax.fori_loop(0, _CACHE_FRAMES, init, 0, unroll=8)

    def setmap(i, c):
        src_s[idx_s[i]] = i
        return c
    lax.fori_loop(0, _NEW_FRAMES, setmap, 0, unroll=8)

    def start_in(new_h, cache_h, buf, sems, w, b):
        s = src_s[w]

        @pl.when(s >= 0)
        def _():
            pltpu.make_async_copy(new_h.at[s], buf.at[b], sems.at[b]).start()

        @pl.when(s < 0)
        def _():
            pltpu.make_async_copy(cache_h.at[w], buf.at[b], sems.at[b]).start()

    def prime(w, c):
        start_in(kf_h, kc_h, kbuf, kin_sems, w, w % _NBUF)
        start_in(vf_h, vc_h, vbuf, vin_sems, w, w % _NBUF)
        return c
    lax.fori_loop(0, _LOOKAHEAD, prime, 0)

    def step(w, c):
        b = w % _NBUF
        pltpu.make_async_copy(kbuf.at[b], ok_h.at[w], kin_sems.at[b]).wait()
        pltpu.make_async_copy(kbuf.at[b], ok_h.at[w], kout_sems.at[b]).start()
        pltpu.make_async_copy(vbuf.at[b], ov_h.at[w], vin_sems.at[b]).wait()
        pltpu.make_async_copy(vbuf.at[b], ov_h.at[w], vout_sems.at[b]).start()

        u = w + _LOOKAHEAD

        @pl.when(u < _CACHE_FRAMES)
        def _():
            bu = u % _NBUF

            @pl.when(u >= _NBUF)
            def _():
                pltpu.make_async_copy(kbuf.at[bu], ok_h.at[0], kout_sems.at[bu]).wait()
                pltpu.make_async_copy(vbuf.at[bu], ov_h.at[0], vout_sems.at[bu]).wait()

            start_in(kf_h, kc_h, kbuf, kin_sems, u, bu)
            start_in(vf_h, vc_h, vbuf, vin_sems, u, bu)
        return c
    lax.fori_loop(0, _CACHE_FRAMES, step, 0)

    def drain(b, c):
        pltpu.make_async_copy(kbuf.at[b], ok_h.at[0], kout_sems.at[b]).wait()
        pltpu.make_async_copy(vbuf.at[b], ov_h.at[0], vout_sems.at[b]).wait()
        return c
    lax.fori_loop(0, _NBUF, drain, 0)


def kernel(k, v, idx, k_cache, v_cache):
    idx32 = idx.astype(jnp.int32) % _CACHE_FRAMES
    out_k, out_v = pl.pallas_call(
        _store_body,
        out_shape=(
            jax.ShapeDtypeStruct((_CACHE_FRAMES, _TOK, _D), k_cache.dtype),
            jax.ShapeDtypeStruct((_CACHE_FRAMES, _TOK, _D), v_cache.dtype),
        ),
        in_specs=[
            pl.BlockSpec(memory_space=_SMEM),
            pl.BlockSpec(memory_space=_ANY),
            pl.BlockSpec(memory_space=_ANY),
            pl.BlockSpec(memory_space=_ANY),
            pl.BlockSpec(memory_space=_ANY),
        ],
        out_specs=(
            pl.BlockSpec(memory_space=_ANY),
            pl.BlockSpec(memory_space=_ANY),
        ),
        scratch_shapes=[
            pltpu.SMEM((_CACHE_FRAMES,), jnp.int32),
            pltpu.VMEM((_NBUF, _TOK, _D), jnp.float32),
            pltpu.VMEM((_NBUF, _TOK, _D), jnp.float32),
            pltpu.SemaphoreType.DMA((_NBUF,)),
            pltpu.SemaphoreType.DMA((_NBUF,)),
            pltpu.SemaphoreType.DMA((_NBUF,)),
            pltpu.SemaphoreType.DMA((_NBUF,)),
        ],
    )(idx32,
      k.reshape(_NEW_FRAMES, _TOK, _D),
      v.reshape(_NEW_FRAMES, _TOK, _D),
      k_cache.reshape(_CACHE_FRAMES, _TOK, _D),
      v_cache.reshape(_CACHE_FRAMES, _TOK, _D))
    return out_k.reshape(k_cache.shape), out_v.reshape(v_cache.shape)


# speedup vs baseline: 24.6141x; 1.0023x over previous
import jax
import jax.numpy as jnp
from jax import lax
from jax.experimental import pallas as pl
from jax.experimental.pallas import tpu as pltpu

_CACHE_FRAMES = 256
_NEW_FRAMES = 64
_TOK = 128
_D = 16 * 64

_NBUF = 16
_LOOKAHEAD = 12

_ANY = pl.ANY
_SMEM = pltpu.MemorySpace.SMEM


def _store_body(idx_s, kf_h, vf_h, kc_h, vc_h, ok_h, ov_h,
                src_s, kbuf, vbuf, in_sems, out_sems):
    def init(j, c):
        src_s[j] = -1
        return c
    lax.fori_loop(0, _CACHE_FRAMES, init, 0, unroll=8)

    def setmap(i, c):
        src_s[idx_s[i]] = i
        return c
    lax.fori_loop(0, _NEW_FRAMES, setmap, 0, unroll=8)

    def start_in(w, b):
        s = src_s[w]

        @pl.when(s >= 0)
        def _():
            pltpu.make_async_copy(kf_h.at[s], kbuf.at[b], in_sems.at[b]).start()
            pltpu.make_async_copy(vf_h.at[s], vbuf.at[b], in_sems.at[b]).start()

        @pl.when(s < 0)
        def _():
            pltpu.make_async_copy(kc_h.at[w], kbuf.at[b], in_sems.at[b]).start()
            pltpu.make_async_copy(vc_h.at[w], vbuf.at[b], in_sems.at[b]).start()

    def wait_pair(sems, b):
        pltpu.make_async_copy(
            kc_h.at[pl.ds(0, 2)], kbuf.at[pl.ds(0, 2)], sems.at[b]).wait()

    def prime(w, c):
        start_in(w, w % _NBUF)
        return c
    lax.fori_loop(0, _LOOKAHEAD, prime, 0)

    def step(w, c):
        b = w % _NBUF
        wait_pair(in_sems, b)
        pltpu.make_async_copy(kbuf.at[b], ok_h.at[w], out_sems.at[b]).start()
        pltpu.make_async_copy(vbuf.at[b], ov_h.at[w], out_sems.at[b]).start()

        u = w + _LOOKAHEAD

        @pl.when(u < _CACHE_FRAMES)
        def _():
            bu = u % _NBUF

            @pl.when(u >= _NBUF)
            def _():
                wait_pair(out_sems, bu)

            start_in(u, bu)
        return c
    lax.fori_loop(0, _CACHE_FRAMES, step, 0, unroll=2)

    def drain(b, c):
        wait_pair(out_sems, b)
        return c
    lax.fori_loop(0, _NBUF, drain, 0)


def kernel(k, v, idx, k_cache, v_cache):
    idx32 = idx.astype(jnp.int32) % _CACHE_FRAMES
    out_k, out_v = pl.pallas_call(
        _store_body,
        out_shape=(
            jax.ShapeDtypeStruct((_CACHE_FRAMES, _TOK, _D), k_cache.dtype),
            jax.ShapeDtypeStruct((_CACHE_FRAMES, _TOK, _D), v_cache.dtype),
        ),
        in_specs=[
            pl.BlockSpec(memory_space=_SMEM),
            pl.BlockSpec(memory_space=_ANY),
            pl.BlockSpec(memory_space=_ANY),
            pl.BlockSpec(memory_space=_ANY),
            pl.BlockSpec(memory_space=_ANY),
        ],
        out_specs=(
            pl.BlockSpec(memory_space=_ANY),
            pl.BlockSpec(memory_space=_ANY),
        ),
        scratch_shapes=[
            pltpu.SMEM((_CACHE_FRAMES,), jnp.int32),
            pltpu.VMEM((_NBUF, _TOK, _D), jnp.float32),
            pltpu.VMEM((_NBUF, _TOK, _D), jnp.float32),
            pltpu.SemaphoreType.DMA((_NBUF,)),
            pltpu.SemaphoreType.DMA((_NBUF,)),
        ],
    )(idx32,
      k.reshape(_NEW_FRAMES, _TOK, _D),
      v.reshape(_NEW_FRAMES, _TOK, _D),
      k_cache.reshape(_CACHE_FRAMES, _TOK, _D),
      v_cache.reshape(_CACHE_FRAMES, _TOK, _D))
    return out_k.reshape(k_cache.shape), out_v.reshape(v_cache.shape)
